# Initial kernel scaffold; baseline (speedup 1.0000x reference)
import jax
import jax.numpy as jnp
from jax.experimental import pallas as pl


def kernel(X, table):
    raise NotImplementedError("write your pallas kernel here")


# speedup vs baseline: 1.1035x; 1.1035x over previous
import functools

import jax
import jax.numpy as jnp
from jax import lax
from jax.experimental import pallas as pl
from jax.experimental.pallas import tpu as pltpu
from jax.experimental.pallas import tpu_sc as plsc


@functools.partial(jax.jit, static_argnums=(2, 3))
def _sc_gather(table, idx, N, D):
    info = plsc.get_sparse_core_info()
    NC, NS = info.num_cores, info.num_subcores
    NW = NC * NS
    n_per_w = N // NW
    C = 1600
    n_chunks = n_per_w // C

    mesh = plsc.VectorSubcoreMesh(core_axis_name="c", subcore_axis_name="s")

    @functools.partial(
        pl.kernel,
        mesh=mesh,
        out_type=jax.ShapeDtypeStruct((N, D), jnp.float32),
        scratch_types=[
            pltpu.VMEM((C,), jnp.int32),
            pltpu.VMEM((C, D), jnp.float32),
            pltpu.SemaphoreType.DMA,
        ],
        compiler_params=pltpu.CompilerParams(use_tc_tiling_on_sc=False),
    )
    def k(table_hbm, idx_hbm, out_hbm, idx_v, rows_v, sem):
        wid = lax.axis_index("s") * NC + lax.axis_index("c")
        base = wid * n_per_w

        def body(g, carry):
            off = base + g * C
            pltpu.sync_copy(idx_hbm.at[pl.ds(off, C)], idx_v)
            pltpu.async_copy(table_hbm.at[idx_v], rows_v, sem).wait()
            pltpu.sync_copy(rows_v, out_hbm.at[pl.ds(off, C)])
            return carry

        lax.fori_loop(0, n_chunks, body, 0)

    return k(table, idx)


def kernel(X, table):
    B, H = X.shape
    V, D = table.shape
    N = B * H
    idx = X.reshape(N).astype(jnp.int32)
    out = _sc_gather(table, idx, N, D)
    return out.reshape(B, H, D)


# speedup vs baseline: 1.1135x; 1.0091x over previous
import functools

import jax
import jax.numpy as jnp
from jax import lax
from jax.experimental import pallas as pl
from jax.experimental.pallas import tpu as pltpu
from jax.experimental.pallas import tpu_sc as plsc


@functools.partial(jax.jit, static_argnums=(2, 3))
def _sc_gather(table, idx, N, D):
    info = plsc.get_sparse_core_info()
    NC, NS = info.num_cores, info.num_subcores
    NW = NC * NS
    n_per_w = N // NW
    C = 1600
    n_chunks = n_per_w // C

    mesh = plsc.VectorSubcoreMesh(core_axis_name="c", subcore_axis_name="s")

    @functools.partial(
        pl.kernel,
        mesh=mesh,
        out_type=jax.ShapeDtypeStruct((N, D), jnp.float32),
        scratch_types=[
            pltpu.VMEM((n_per_w,), jnp.int32),
            pltpu.VMEM((C, D), jnp.float32),
            pltpu.VMEM((C, D), jnp.float32),
            pltpu.SemaphoreType.DMA,
            pltpu.SemaphoreType.DMA,
            pltpu.SemaphoreType.DMA,
            pltpu.SemaphoreType.DMA,
        ],
        compiler_params=pltpu.CompilerParams(use_tc_tiling_on_sc=False),
    )
    def k(table_hbm, idx_hbm, out_hbm, idx_v, rows0, rows1, sg0, sg1, so0, so1):
        wid = lax.axis_index("s") * NC + lax.axis_index("c")
        base = wid * n_per_w
        pltpu.sync_copy(idx_hbm.at[pl.ds(base, n_per_w)], idx_v)

        rows = [rows0, rows1]
        sg = [sg0, sg1]
        so = [so0, so1]
        gathers = [None, None]
        writes = [None, None]
        gathers[0] = pltpu.async_copy(
            table_hbm.at[idx_v.at[pl.ds(0, C)]], rows[0], sg[0])
        for g in range(n_chunks):
            b = g & 1
            nb = 1 - b
            if g + 1 < n_chunks:
                if writes[nb] is not None:
                    writes[nb].wait()
                gathers[nb] = pltpu.async_copy(
                    table_hbm.at[idx_v.at[pl.ds((g + 1) * C, C)]],
                    rows[nb], sg[nb])
            gathers[b].wait()
            writes[b] = pltpu.async_copy(
                rows[b], out_hbm.at[pl.ds(base + g * C, C)], so[b])
        writes[0].wait()
        writes[1].wait()

    return k(table, idx)


def kernel(X, table):
    B, H = X.shape
    V, D = table.shape
    N = B * H
    idx = X.reshape(N).astype(jnp.int32)
    out = _sc_gather(table, idx, N, D)
    return out.reshape(B, H, D)


# speedup vs baseline: 1.4792x; 1.3284x over previous
import functools

import jax
import jax.numpy as jnp
from jax import lax
from jax.experimental import pallas as pl
from jax.experimental.pallas import tpu as pltpu
from jax.experimental.pallas import tpu_sc as plsc


@functools.partial(jax.jit, static_argnums=(2, 3, 4))
def _sc_gather_t(table, idx, B, H, D):
    info = plsc.get_sparse_core_info()
    NC, NS = info.num_cores, info.num_subcores
    NW = NC * NS
    R = D // 8
    CB = B // 128
    G = CB // 4
    UNITS = H * G
    UPW = UNITS // NW
    C = 512

    mesh = plsc.VectorSubcoreMesh(core_axis_name="c", subcore_axis_name="s")

    @functools.partial(
        pl.kernel,
        mesh=mesh,
        out_type=jax.ShapeDtypeStruct((B * H * D,), jnp.float32),
        scratch_types=[
            pltpu.VMEM((C,), jnp.int32),
            pltpu.VMEM((C, D), jnp.float32),
            pltpu.VMEM((C * D,), jnp.float32),
            pltpu.SemaphoreType.DMA,
            pltpu.SemaphoreType.DMA,
        ],
        compiler_params=pltpu.CompilerParams(
            use_tc_tiling_on_sc=False, needs_layout_passes=False),
    )
    def k(table_hbm, idx_hbm, out_hbm, idx_v, rows_v, trows, sg, sw):
        w = lax.axis_index("s") * NC + lax.axis_index("c")
        iota = lax.iota(jnp.int32, 16)

        def unit(t, carry):
            u = w * UPW + t
            h = u >> 5
            g = u & (G - 1)
            pltpu.sync_copy(idx_hbm.at[pl.ds(h * B + g * C, C)], idx_v)
            pltpu.async_copy(table_hbm.at[idx_v], rows_v, sg).wait()

            def mbody(m, c2):
                r = m >> 2
                cb = m & 3
                for i in range(8):
                    col = jnp.full((16,), 8 * r + i, jnp.int32)
                    for j0 in range(0, 128, 16):
                        row_ids = iota + (cb * 128 + j0)
                        vals = plsc.load_gather(rows_v, [row_ids, col])
                        t0 = r * 4096 + cb * 1024 + i * 128 + j0
                        trows[pl.ds(t0, 16)] = vals
                return c2

            lax.fori_loop(0, 16, mbody, 0, unroll=4)

            copies = []
            for r in range(R):
                flat0 = h * (R * CB * 1024) + r * (CB * 1024) + g * 4096
                copies.append(pltpu.async_copy(
                    trows.at[pl.ds(r * 4096, 4096)],
                    out_hbm.at[pl.ds(flat0, 4096)], sw))
            for cp in copies:
                cp.wait()
            return carry

        lax.fori_loop(0, UPW, unit, 0)

    return k(table, idx)


def kernel(X, table):
    B, H = X.shape
    V, D = table.shape
    idx = X.T.reshape(B * H).astype(jnp.int32)
    out_flat = _sc_gather_t(table, idx, B, H, D)
    R = D // 8
    CB = B // 128
    out = (out_flat.reshape(H, R, CB, 8, 128)
           .transpose(2, 4, 0, 1, 3)
           .reshape(B, H, D))
    return out


# speedup vs baseline: 1.8671x; 1.2622x over previous
import functools

import jax
import jax.numpy as jnp
from jax import lax
from jax.experimental import pallas as pl
from jax.experimental.pallas import tpu as pltpu
from jax.experimental.pallas import tpu_sc as plsc


@functools.partial(jax.jit, static_argnums=(2, 3, 4))
def _sc_gather_t(table, idx, B, H, D):
    info = plsc.get_sparse_core_info()
    NC, NS = info.num_cores, info.num_subcores
    NW = NC * NS
    R = D // 8
    CB = B // 128
    G = CB // 4
    UNITS = H * G
    UPW = UNITS // NW
    C = 512
    HSTRIDE = R * CB * 1024
    RSTRIDE = CB * 1024

    mesh = plsc.VectorSubcoreMesh(core_axis_name="c", subcore_axis_name="s")

    @functools.partial(
        pl.kernel,
        mesh=mesh,
        out_type=jax.ShapeDtypeStruct((B * H * D,), jnp.float32),
        scratch_types=[
            pltpu.VMEM((UPW * C,), jnp.int32),
            pltpu.VMEM((C, D), jnp.float32),
            pltpu.VMEM((C, D), jnp.float32),
            pltpu.VMEM((C * D,), jnp.float32),
            pltpu.VMEM((C * D,), jnp.float32),
            pltpu.SemaphoreType.DMA,
            pltpu.SemaphoreType.DMA,
            pltpu.SemaphoreType.DMA,
            pltpu.SemaphoreType.DMA,
        ],
        compiler_params=pltpu.CompilerParams(
            use_tc_tiling_on_sc=False, needs_layout_passes=False),
    )
    def k(table_hbm, idx_hbm, out_hbm, idx_all, rows0, rows1, tr0, tr1,
          sg0, sg1, sw0, sw1):
        w = lax.axis_index("s") * NC + lax.axis_index("c")
        base_u = w * UPW
        pltpu.sync_copy(idx_hbm.at[pl.ds(w * (UPW * C), UPW * C)], idx_all)

        iota = lax.iota(jnp.int32, 16)
        v_lo = ((iota >> 3) << 12) + ((iota & 7) << 7)
        v_hi = v_lo + 2 * 4096

        rows = [rows0, rows1]
        trs = [tr0, tr1]
        sg = [sg0, sg1]
        sw = [sw0, sw1]

        def gather_desc(t, b):
            return pltpu.make_async_copy(
                table_hbm.at[idx_all.at[pl.ds(t * C, C)]], rows[b], sg[b])

        def write_descs(t, b):
            u = base_u + t
            h = u >> 5
            g = u & (G - 1)
            ds_ = []
            for r in range(R):
                flat0 = h * HSTRIDE + r * RSTRIDE + g * 4096
                ds_.append(pltpu.make_async_copy(
                    trs[b].at[pl.ds(r * 4096, 4096)],
                    out_hbm.at[pl.ds(flat0, 4096)], sw[b]))
            return ds_

        def transpose_unit(rows_b, tr_b):
            def body(i8, carry):
                for kk in range(8):
                    q = i8 * 8 + kk
                    const = ((q >> 7) << 10) + (q & 127)
                    v0 = rows_b[q, pl.ds(0, 16)]
                    v1 = rows_b[q, pl.ds(16, 16)]
                    plsc.store_scatter(tr_b, [v_lo + const], v0)
                    plsc.store_scatter(tr_b, [v_hi + const], v1)
                return carry
            lax.fori_loop(0, C // 8, body, 0, unroll=4)

        gather_desc(0, 0).start()

        def step(s, carry):
            for b in range(2):
                t = 2 * s + b
                nb = 1 - b
                @pl.when(t + 1 < UPW)
                def _():
                    gather_desc(t + 1, nb).start()
                gather_desc(t, b).wait()
                @pl.when(t >= 2)
                def _():
                    for d_ in write_descs(t - 2, b):
                        d_.wait()
                transpose_unit(rows[b], trs[b])
                for d_ in write_descs(t, b):
                    d_.start()
            return carry

        lax.fori_loop(0, UPW // 2, step, 0)
        for d_ in write_descs(UPW - 2, 0):
            d_.wait()
        for d_ in write_descs(UPW - 1, 1):
            d_.wait()

    return k(table, idx)


def kernel(X, table):
    B, H = X.shape
    V, D = table.shape
    idx = X.T.reshape(B * H).astype(jnp.int32)
    out_flat = _sc_gather_t(table, idx, B, H, D)
    R = D // 8
    CB = B // 128
    out = (out_flat.reshape(H, R, CB, 8, 128)
           .transpose(2, 4, 0, 1, 3)
           .reshape(B, H, D))
    return out


# speedup vs baseline: 2.5381x; 1.3594x over previous
import functools

import jax
import jax.numpy as jnp
from jax import lax
from jax.experimental import pallas as pl
from jax.experimental.pallas import tpu as pltpu
from jax.experimental.pallas import tpu_sc as plsc


@functools.partial(jax.jit, static_argnums=(2, 3, 4))
def _sc_gather_t(table, idx, B, H, D):
    info = plsc.get_sparse_core_info()
    NC, NS = info.num_cores, info.num_subcores
    NW = NC * NS
    R = D // 8
    CB = B // 128
    G = CB // 4
    UNITS = H * G
    UPW = UNITS // NW
    C = 512
    HSTRIDE = R * CB * 1024
    RSTRIDE = CB * 1024

    mesh = plsc.VectorSubcoreMesh(core_axis_name="c", subcore_axis_name="s")

    @functools.partial(
        pl.kernel,
        mesh=mesh,
        out_type=jax.ShapeDtypeStruct((B * H * D,), jnp.float32),
        scratch_types=[
            pltpu.VMEM((UPW * C,), jnp.int32),
            pltpu.VMEM((C, D), jnp.float32),
            pltpu.VMEM((C, D), jnp.float32),
            pltpu.VMEM((C * D,), jnp.float32),
            pltpu.VMEM((C * D,), jnp.float32),
            pltpu.SemaphoreType.DMA,
            pltpu.SemaphoreType.DMA,
            pltpu.SemaphoreType.DMA,
            pltpu.SemaphoreType.DMA,
        ],
        compiler_params=pltpu.CompilerParams(
            use_tc_tiling_on_sc=False, needs_layout_passes=False),
    )
    def k(table_hbm, idx_hbm, out_hbm, idx_all, rows0, rows1, tr0, tr1,
          sg0, sg1, sw0, sw1):
        w = lax.axis_index("s") * NC + lax.axis_index("c")
        base_u = w * UPW
        pltpu.sync_copy(idx_hbm.at[pl.ds(w * (UPW * C), UPW * C)], idx_all)

        iota = lax.iota(jnp.int32, 16)

        rows = [rows0, rows1]
        trs = [tr0, tr1]
        sg = [sg0, sg1]
        sw = [sw0, sw1]

        def gather_desc(t, b):
            return pltpu.make_async_copy(
                table_hbm.at[idx_all.at[pl.ds(t * C, C)]], rows[b], sg[b])

        def write_descs(t, b):
            u = base_u + t
            h = u >> 5
            g = u & (G - 1)
            ds_ = []
            for r in range(R):
                flat0 = h * HSTRIDE + r * RSTRIDE + g * 4096
                ds_.append(pltpu.make_async_copy(
                    trs[b].at[pl.ds(r * 4096, 4096)],
                    out_hbm.at[pl.ds(flat0, 4096)], sw[b]))
            return ds_

        def transpose_unit(rows_b, tr_b):
            def abody(a, carry):
                colv = (iota + a) & 31
                storepat = ((colv >> 3) << 12) + ((colv & 7) << 7) + iota

                def body(blk, c2):
                    row_ids = iota + (blk << 4)
                    vals = plsc.load_gather(rows_b, [row_ids, colv])
                    sconst = ((blk >> 3) << 10) + ((blk & 7) << 4)
                    plsc.store_scatter(tr_b, [storepat + sconst], vals)
                    return c2

                lax.fori_loop(0, C // 16, body, 0, unroll=4)
                return carry

            lax.fori_loop(0, D, abody, 0)

        gather_desc(0, 0).start()

        def step(s, carry):
            for b in range(2):
                t = 2 * s + b
                nb = 1 - b
                @pl.when(t + 1 < UPW)
                def _():
                    gather_desc(t + 1, nb).start()
                gather_desc(t, b).wait()
                @pl.when(t >= 2)
                def _():
                    for d_ in write_descs(t - 2, b):
                        d_.wait()
                transpose_unit(rows[b], trs[b])
                for d_ in write_descs(t, b):
                    d_.start()
            return carry

        lax.fori_loop(0, UPW // 2, step, 0)
        for d_ in write_descs(UPW - 2, 0):
            d_.wait()
        for d_ in write_descs(UPW - 1, 1):
            d_.wait()

    return k(table, idx)


def kernel(X, table):
    B, H = X.shape
    V, D = table.shape
    idx = X.T.reshape(B * H).astype(jnp.int32)
    out_flat = _sc_gather_t(table, idx, B, H, D)
    R = D // 8
    CB = B // 128
    out = (out_flat.reshape(H, R, CB, 8, 128)
           .transpose(2, 4, 0, 1, 3)
           .reshape(B, H, D))
    return out
